# baseline (speedup 1.0000x reference)
import functools

import jax
import jax.numpy as jnp
from jax import lax
from jax.experimental import pallas as pl
from jax.experimental.pallas import tpu as pltpu
from jax.experimental.pallas import tpu_sc as plsc

NUM_USERS = 1000000
NUM_ITEMS = 1000000
EMB_DIM = 32
BATCH = 16384

ROWS_PER_LINE = 4
LINE = ROWS_PER_LINE * EMB_DIM

NC = 2
NS = 16
NW = NC * NS
BPW = BATCH // NW
CHUNK = 128
LANES = 16
PASS = 256
NPASS = BPW // PASS
PASS_CHUNKS = PASS // CHUNK
PASS_GROUPS = PASS // LANES

_mesh = plsc.VectorSubcoreMesh(
    core_axis_name="c", subcore_axis_name="s", num_cores=NC, num_subcores=NS
)


@functools.partial(
    pl.kernel,
    out_type=jax.ShapeDtypeStruct((BATCH,), jnp.float32),
    mesh=_mesh,
    scratch_types=[
        pltpu.VMEM((BPW,), jnp.int32),
        pltpu.VMEM((BPW,), jnp.int32),
        pltpu.VMEM((BPW,), jnp.int32),
        pltpu.VMEM((BPW,), jnp.int32),
        pltpu.VMEM((PASS, LINE), jnp.float32),
        pltpu.VMEM((PASS, LINE), jnp.float32),
        pltpu.VMEM((BPW,), jnp.float32),
        pltpu.SemaphoreType.DMA,
    ],
    compiler_params=pltpu.CompilerParams(
        needs_layout_passes=False, use_tc_tiling_on_sc=True),
)
def _lightgcn_sc(uid_hbm, iid_hbm, utab_hbm, itab_hbm, out_hbm,
                 uidx_v, iidx_v, uline_v, iline_v, urows_v, irows_v,
                 out_v, sem):
    wid = lax.axis_index("s") * NC + lax.axis_index("c")
    base = wid * BPW

    pltpu.sync_copy(uid_hbm.at[pl.ds(base, BPW)], uidx_v)
    pltpu.sync_copy(iid_hbm.at[pl.ds(base, BPW)], iidx_v)

    def line_body(k, carry):
        sl = pl.ds(k * LANES, LANES)
        uline_v[sl] = lax.shift_right_logical(uidx_v[sl], 2)
        iline_v[sl] = lax.shift_right_logical(iidx_v[sl], 2)
        return carry

    lax.fori_loop(0, BPW // LANES, line_body, None)

    lane_iota = lax.iota(jnp.int32, LANES)

    for p in range(NPASS):
        copies = []
        for j in range(PASS_CHUNKS):
            src = pl.ds(p * PASS + j * CHUNK, CHUNK)
            dst = pl.ds(j * CHUNK, CHUNK)
            copies.append(
                pltpu.async_copy(utab_hbm.at[uline_v.at[src]],
                                 urows_v.at[dst], sem))
            copies.append(
                pltpu.async_copy(itab_hbm.at[iline_v.at[src]],
                                 irows_v.at[dst], sem))
        for cp in copies:
            cp.wait()

        def group_body(g, carry):
            row0 = g * LANES
            row_idx = row0 + lane_iota
            gsl = pl.ds(p * PASS + row0, LANES)
            ucol0 = lax.shift_left(jnp.bitwise_and(uidx_v[gsl], 3), 5)
            icol0 = lax.shift_left(jnp.bitwise_and(iidx_v[gsl], 3), 5)
            acc = jnp.zeros((LANES,), jnp.float32)
            for d in range(EMB_DIM):
                u = plsc.load_gather(urows_v, [row_idx, ucol0 + d])
                v = plsc.load_gather(irows_v, [row_idx, icol0 + d])
                acc = acc + u * v
            out_v[gsl] = acc
            return carry

        lax.fori_loop(0, PASS_GROUPS, group_body, None)

    pltpu.sync_copy(out_v, out_hbm.at[pl.ds(base, BPW)])


def kernel(user_ids, item_ids, user_embeddings, item_embeddings):
    return _lightgcn_sc(
        user_ids.astype(jnp.int32),
        item_ids.astype(jnp.int32),
        user_embeddings.reshape(NUM_USERS // ROWS_PER_LINE, LINE),
        item_embeddings.reshape(NUM_ITEMS // ROWS_PER_LINE, LINE),
    )
